# baseline (speedup 1.0000x reference)
import jax
import jax.numpy as jnp
from jax import lax
from jax.experimental import pallas as pl
from jax.experimental.pallas import tpu as pltpu
from jax.experimental.pallas import tpu_sc as plsc

N = 10000
E = 320000
D = 128
DH = 256

NC = 2
NS = 16
NW = NC * NS
C = 128
CH = 80
PH = 16
NPH = CH // PH
EPW = C * CH
EPAD = EPW * NW
PAD = EPAD - E
NACC = 10240
RPT = NACC // NS
NPAD = N + 8


def _seg_sum_body(table, src, dst, zeros, out, src_v, dst_v, rows_v, acc, gsem, ssem):
    c = lax.axis_index("c")
    s = lax.axis_index("s")
    wid = c * NS + s
    pltpu.sync_copy(zeros, acc.at[pl.ds(s * RPT, RPT)])
    plsc.subcore_barrier()

    def wait_rows(sem):
        pltpu.make_async_copy(table.at[pl.ds(0, C)], rows_v.at[0], sem).wait()

    for ph in range(NPH):
        pltpu.sync_copy(src.at[wid, pl.ds(ph * PH, PH)], src_v)
        pltpu.sync_copy(dst.at[wid, pl.ds(ph * PH, PH)], dst_v)
        pltpu.async_copy(table.at[src_v.at[0]], rows_v.at[0], gsem)

        def body(k0, carry):
            j0 = k0 * 2
            for b in range(2):
                j = j0 + b

                @pl.when(j + 1 < PH)
                def _():
                    pltpu.async_copy(
                        table.at[src_v.at[j + 1]], rows_v.at[1 - b], gsem
                    )

                wait_rows(gsem)
                pltpu.sync_copy(rows_v.at[b], acc.at[dst_v.at[j]], add=True)
            return carry

        lax.fori_loop(0, PH // 2, body, 0)

    plsc.subcore_barrier()
    pltpu.sync_copy(acc.at[pl.ds(s * RPT, RPT)], out.at[c, pl.ds(s * RPT, RPT)])


import functools


@functools.cache
def _make_seg_sum():
    return pl.kernel(
        _seg_sum_body,
        out_type=jax.ShapeDtypeStruct((NC, NACC, D), jnp.float32),
        mesh=plsc.VectorSubcoreMesh(
            core_axis_name="c", subcore_axis_name="s", num_cores=NC, num_subcores=NS
        ),
        scratch_types=[
            pltpu.VMEM((PH, C), jnp.int32),
            pltpu.VMEM((PH, C), jnp.int32),
            pltpu.VMEM((2, C, D), jnp.float32),
            pltpu.VMEM_SHARED((NACC, D), jnp.float32),
            pltpu.SemaphoreType.DMA,
            pltpu.SemaphoreType.DMA,
        ],
    )


RB = 400
GRID = N // RB


def _dense1_body(p0, p1, x, wl1, wr1, b1, g, b, wl2, wr2, b2, y2, r2):
    agg = p0[...] + p1[...]
    h = jnp.dot(agg, wl1[...], preferred_element_type=jnp.float32)
    h = h + jnp.dot(x[...], wr1[...], preferred_element_type=jnp.float32)
    h = h + b1[...]
    mu = jnp.mean(h, axis=-1, keepdims=True)
    var = jnp.mean((h - mu) ** 2, axis=-1, keepdims=True)
    hn = (h - mu) * lax.rsqrt(var + 1e-5) * g[...] + b[...]
    hr = jnp.maximum(hn, 0.0)
    y2[...] = jnp.dot(hr, wl2[...], preferred_element_type=jnp.float32)
    r2[...] = jnp.dot(hr, wr2[...], preferred_element_type=jnp.float32) + b2[...]


def _combine_body(a, b, c, o):
    o[...] = a[...] + b[...] + c[...]


def _dense1(p0, p1, x, W_l1, W_r1, b1r, gr, br, W_l2, W_r2, b2r):
    blk = lambda r, d: pl.BlockSpec((r, d), lambda i: (i, 0))
    full = lambda a, d: pl.BlockSpec((a, d), lambda i: (0, 0))
    return pl.pallas_call(
        _dense1_body,
        grid=(GRID,),
        in_specs=[
            blk(RB, D), blk(RB, D), blk(RB, D),
            full(D, DH), full(D, DH), full(1, DH), full(1, DH), full(1, DH),
            full(DH, D), full(DH, D), full(1, D),
        ],
        out_specs=[blk(RB, D), blk(RB, D)],
        out_shape=[
            jax.ShapeDtypeStruct((N, D), jnp.float32),
            jax.ShapeDtypeStruct((N, D), jnp.float32),
        ],
    )(p0, p1, x, W_l1, W_r1, b1r, gr, br, W_l2, W_r2, b2r)


def _combine(q0, q1, r2):
    blk = pl.BlockSpec((RB, D), lambda i: (i, 0))
    return pl.pallas_call(
        _combine_body,
        grid=(GRID,),
        in_specs=[blk, blk, blk],
        out_specs=blk,
        out_shape=jax.ShapeDtypeStruct((N, D), jnp.float32),
    )(q0, q1, r2)


def kernel(x, edge_index, W_l1, W_r1, b1, ln_g, ln_b, W_l2, W_r2, b2):
    f32 = jnp.float32
    i32 = jnp.int32
    src = jnp.concatenate([edge_index[0], jnp.full((PAD,), N, i32)]).reshape(NW, CH, C)
    pad_dst = (jnp.arange(PAD, dtype=i32) * 8) % N
    dst = jnp.concatenate([edge_index[1], pad_dst]).reshape(NW, CH, C)
    zeros = jnp.zeros((RPT, D), f32)
    xp = jnp.concatenate([x, jnp.zeros((NPAD - N, D), f32)], axis=0)

    seg_sum = _make_seg_sum()
    p = seg_sum(xp, src, dst, zeros)[:, :N]

    b1r = b1.reshape(1, DH)
    gr = ln_g.reshape(1, DH)
    br = ln_b.reshape(1, DH)
    b2r = b2.reshape(1, D)
    y2, r2 = _dense1(p[0], p[1], x, W_l1, W_r1, b1r, gr, br, W_l2, W_r2, b2r)

    y2p = jnp.concatenate([y2, jnp.zeros((NPAD - N, D), f32)], axis=0)
    q = seg_sum(y2p, src, dst, zeros)[:, :N]

    return _combine(q[0], q[1], r2)


# speedup vs baseline: 3.4537x; 3.4537x over previous
import jax
import jax.numpy as jnp
from jax import lax
from jax.experimental import pallas as pl
from jax.experimental.pallas import tpu as pltpu
from jax.experimental.pallas import tpu_sc as plsc

N = 10000
E = 320000
D = 128
DH = 256

NC = 2
NS = 16
NW = NC * NS
C = 128
CH = 80
PH = 16
NPH = CH // PH
EPW = C * CH
EPAD = EPW * NW
PAD = EPAD - E
NACC = 10240
RPT = NACC // NS
NPAD = N + 128


def _seg_sum_body(table, src, dst, zeros, out, src_v, dst_v, rows_v, acc, gsem, ssem):
    c = lax.axis_index("c")
    s = lax.axis_index("s")
    wid = c * NS + s
    pltpu.sync_copy(zeros, acc.at[pl.ds(s * RPT, RPT)])
    plsc.subcore_barrier()

    def wait_rows(sem):
        pltpu.make_async_copy(table.at[pl.ds(0, C)], rows_v.at[0], sem).wait()

    for ph in range(NPH):
        pltpu.sync_copy(src.at[wid, pl.ds(ph * PH, PH)], src_v)
        pltpu.sync_copy(dst.at[wid, pl.ds(ph * PH, PH)], dst_v)
        pltpu.async_copy(table.at[src_v.at[0]], rows_v.at[0], gsem)

        def body(k0, carry):
            j0 = k0 * 2
            for b in range(2):
                j = j0 + b

                @pl.when(j + 1 < PH)
                def _():
                    pltpu.async_copy(
                        table.at[src_v.at[j + 1]], rows_v.at[1 - b], gsem
                    )

                wait_rows(gsem)
                pltpu.sync_copy(rows_v.at[b], acc.at[dst_v.at[j]], add=True)
            return carry

        lax.fori_loop(0, PH // 2, body, 0)

    plsc.subcore_barrier()
    pltpu.sync_copy(acc.at[pl.ds(s * RPT, RPT)], out.at[c, pl.ds(s * RPT, RPT)])


import functools


@functools.cache
def _make_seg_sum():
    return pl.kernel(
        _seg_sum_body,
        out_type=jax.ShapeDtypeStruct((NC, NACC, D), jnp.float32),
        mesh=plsc.VectorSubcoreMesh(
            core_axis_name="c", subcore_axis_name="s", num_cores=NC, num_subcores=NS
        ),
        scratch_types=[
            pltpu.VMEM((PH, C), jnp.int32),
            pltpu.VMEM((PH, C), jnp.int32),
            pltpu.VMEM((2, C, D), jnp.float32),
            pltpu.VMEM_SHARED((NACC, D), jnp.float32),
            pltpu.SemaphoreType.DMA,
            pltpu.SemaphoreType.DMA,
        ],
    )


RB = 400
GRID = N // RB


def _dense1_body(p0, p1, x, wl1, wr1, b1, g, b, wl2, wr2, b2, y2, r2):
    agg = p0[...] + p1[...]
    h = jnp.dot(agg, wl1[...], preferred_element_type=jnp.float32)
    h = h + jnp.dot(x[...], wr1[...], preferred_element_type=jnp.float32)
    h = h + b1[...]
    mu = jnp.mean(h, axis=-1, keepdims=True)
    var = jnp.mean((h - mu) ** 2, axis=-1, keepdims=True)
    hn = (h - mu) * lax.rsqrt(var + 1e-5) * g[...] + b[...]
    hr = jnp.maximum(hn, 0.0)
    y2[...] = jnp.dot(hr, wl2[...], preferred_element_type=jnp.float32)
    r2[...] = jnp.dot(hr, wr2[...], preferred_element_type=jnp.float32) + b2[...]


def _combine_body(a, b, c, o):
    o[...] = a[...] + b[...] + c[...]


def _dense1(p0, p1, x, W_l1, W_r1, b1r, gr, br, W_l2, W_r2, b2r):
    blk = lambda r, d: pl.BlockSpec((r, d), lambda i: (i, 0))
    full = lambda a, d: pl.BlockSpec((a, d), lambda i: (0, 0))
    return pl.pallas_call(
        _dense1_body,
        grid=(GRID,),
        in_specs=[
            blk(RB, D), blk(RB, D), blk(RB, D),
            full(D, DH), full(D, DH), full(1, DH), full(1, DH), full(1, DH),
            full(DH, D), full(DH, D), full(1, D),
        ],
        out_specs=[blk(RB, D), blk(RB, D)],
        out_shape=[
            jax.ShapeDtypeStruct((N, D), jnp.float32),
            jax.ShapeDtypeStruct((N, D), jnp.float32),
        ],
    )(p0, p1, x, W_l1, W_r1, b1r, gr, br, W_l2, W_r2, b2r)


def _combine(q0, q1, r2):
    blk = pl.BlockSpec((RB, D), lambda i: (i, 0))
    return pl.pallas_call(
        _combine_body,
        grid=(GRID,),
        in_specs=[blk, blk, blk],
        out_specs=blk,
        out_shape=jax.ShapeDtypeStruct((N, D), jnp.float32),
    )(q0, q1, r2)


def kernel(x, edge_index, W_l1, W_r1, b1, ln_g, ln_b, W_l2, W_r2, b2):
    f32 = jnp.float32
    i32 = jnp.int32
    pad_src = N + (jnp.arange(PAD, dtype=i32) % 128)
    src = jnp.concatenate([edge_index[0], pad_src]).reshape(NW, CH, C)
    pad_dst = (jnp.arange(PAD, dtype=i32) * 8) % N
    dst = jnp.concatenate([edge_index[1], pad_dst]).reshape(NW, CH, C)
    zeros = jnp.zeros((RPT, D), f32)
    xp = jnp.concatenate([x, jnp.zeros((NPAD - N, D), f32)], axis=0)

    seg_sum = _make_seg_sum()
    p = seg_sum(xp, src, dst, zeros)[:, :N]

    b1r = b1.reshape(1, DH)
    gr = ln_g.reshape(1, DH)
    br = ln_b.reshape(1, DH)
    b2r = b2.reshape(1, D)
    y2, r2 = _dense1(p[0], p[1], x, W_l1, W_r1, b1r, gr, br, W_l2, W_r2, b2r)

    y2p = jnp.concatenate([y2, jnp.zeros((NPAD - N, D), f32)], axis=0)
    q = seg_sum(y2p, src, dst, zeros)[:, :N]

    return _combine(q[0], q[1], r2)


# speedup vs baseline: 3.6943x; 1.0697x over previous
import jax
import jax.numpy as jnp
from jax import lax
from jax.experimental import pallas as pl
from jax.experimental.pallas import tpu as pltpu
from jax.experimental.pallas import tpu_sc as plsc

N = 10000
E = 320000
D = 128
DH = 256

NC = 2
NS = 16
NW = NC * NS
C = 128
CH = 80
PH = 16
NPH = CH // PH
EPW = C * CH
EPAD = EPW * NW
PAD = EPAD - E
NACC = 10240
RPT = NACC // NS
NDISC = NACC - N


def _seg_sum_body(table, src, dst, zeros, out, src_v, dst_v, rows_v, acc, gsem, ssem):
    c = lax.axis_index("c")
    s = lax.axis_index("s")
    wid = c * NS + s
    pltpu.sync_copy(zeros, acc.at[pl.ds(s * RPT, RPT)])
    plsc.subcore_barrier()

    def wait_rows(sem):
        pltpu.make_async_copy(table.at[pl.ds(0, C)], rows_v.at[0], sem).wait()

    for ph in range(NPH):
        pltpu.sync_copy(src.at[wid, pl.ds(ph * PH, PH)], src_v)
        pltpu.sync_copy(dst.at[wid, pl.ds(ph * PH, PH)], dst_v)
        pltpu.async_copy(table.at[src_v.at[0]], rows_v.at[0], gsem)

        def body(k0, carry):
            j0 = k0 * 2
            for b in range(2):
                j = j0 + b

                @pl.when(j + 1 < PH)
                def _():
                    pltpu.async_copy(
                        table.at[src_v.at[j + 1]], rows_v.at[1 - b], gsem
                    )

                wait_rows(gsem)
                pltpu.sync_copy(rows_v.at[b], acc.at[dst_v.at[j]], add=True)
            return carry

        lax.fori_loop(0, PH // 2, body, 0)

    plsc.subcore_barrier()
    pltpu.sync_copy(acc.at[pl.ds(s * RPT, RPT)], out.at[c, pl.ds(s * RPT, RPT)])


import functools


@functools.cache
def _make_seg_sum():
    return pl.kernel(
        _seg_sum_body,
        out_type=jax.ShapeDtypeStruct((NC, NACC, D), jnp.float32),
        mesh=plsc.VectorSubcoreMesh(
            core_axis_name="c", subcore_axis_name="s", num_cores=NC, num_subcores=NS
        ),
        scratch_types=[
            pltpu.VMEM((PH, C), jnp.int32),
            pltpu.VMEM((PH, C), jnp.int32),
            pltpu.VMEM((2, C, D), jnp.float32),
            pltpu.VMEM_SHARED((NACC, D), jnp.float32),
            pltpu.SemaphoreType.DMA,
            pltpu.SemaphoreType.DMA,
        ],
    )


RB = 400
GRID = N // RB


def _dense1_body(p0, p1, x, wl1, wr1, b1, g, b, wl2, wr2, b2, y2, r2):
    agg = p0[0] + p1[0]
    h = jnp.dot(agg, wl1[...], preferred_element_type=jnp.float32)
    h = h + jnp.dot(x[...], wr1[...], preferred_element_type=jnp.float32)
    h = h + b1[...]
    mu = jnp.mean(h, axis=-1, keepdims=True)
    var = jnp.mean((h - mu) ** 2, axis=-1, keepdims=True)
    hn = (h - mu) * lax.rsqrt(var + 1e-5) * g[...] + b[...]
    hr = jnp.maximum(hn, 0.0)
    y2[...] = jnp.dot(hr, wl2[...], preferred_element_type=jnp.float32)
    r2[...] = jnp.dot(hr, wr2[...], preferred_element_type=jnp.float32) + b2[...]


def _combine_body(a, b, c, o):
    o[...] = a[0] + b[0] + c[...]


_pblk0 = pl.BlockSpec((1, RB, D), lambda i: (0, i, 0))
_pblk1 = pl.BlockSpec((1, RB, D), lambda i: (1, i, 0))


def _dense1(p, x, W_l1, W_r1, b1r, gr, br, W_l2, W_r2, b2r):
    blk = lambda r, d: pl.BlockSpec((r, d), lambda i: (i, 0))
    full = lambda a, d: pl.BlockSpec((a, d), lambda i: (0, 0))
    return pl.pallas_call(
        _dense1_body,
        grid=(GRID,),
        in_specs=[
            _pblk0, _pblk1, blk(RB, D),
            full(D, DH), full(D, DH), full(1, DH), full(1, DH), full(1, DH),
            full(DH, D), full(DH, D), full(1, D),
        ],
        out_specs=[blk(RB, D), blk(RB, D)],
        out_shape=[
            jax.ShapeDtypeStruct((N, D), jnp.float32),
            jax.ShapeDtypeStruct((N, D), jnp.float32),
        ],
    )(p, p, x, W_l1, W_r1, b1r, gr, br, W_l2, W_r2, b2r)


def _combine(q, r2):
    blk = pl.BlockSpec((RB, D), lambda i: (i, 0))
    return pl.pallas_call(
        _combine_body,
        grid=(GRID,),
        in_specs=[_pblk0, _pblk1, blk],
        out_specs=blk,
        out_shape=jax.ShapeDtypeStruct((N, D), jnp.float32),
    )(q, q, r2)


def kernel(x, edge_index, W_l1, W_r1, b1, ln_g, ln_b, W_l2, W_r2, b2):
    f32 = jnp.float32
    i32 = jnp.int32
    pad_idx = jnp.arange(PAD, dtype=i32)
    pad_src = pad_idx % N
    src = jnp.concatenate([edge_index[0], pad_src]).reshape(NW, CH, C)
    pad_dst = N + pad_idx % NDISC
    dst = jnp.concatenate([edge_index[1], pad_dst]).reshape(NW, CH, C)
    zeros = jnp.zeros((RPT, D), f32)

    seg_sum = _make_seg_sum()
    p = seg_sum(x, src, dst, zeros)

    b1r = b1.reshape(1, DH)
    gr = ln_g.reshape(1, DH)
    br = ln_b.reshape(1, DH)
    b2r = b2.reshape(1, D)
    y2, r2 = _dense1(p, x, W_l1, W_r1, b1r, gr, br, W_l2, W_r2, b2r)

    q = seg_sum(y2, src, dst, zeros)

    return _combine(q, r2)


# speedup vs baseline: 3.8279x; 1.0362x over previous
import jax
import jax.numpy as jnp
from jax import lax
from jax.experimental import pallas as pl
from jax.experimental.pallas import tpu as pltpu
from jax.experimental.pallas import tpu_sc as plsc

N = 10000
E = 320000
D = 128
DH = 256

NC = 2
NS = 16
NW = NC * NS
C = 128
CH = 80
PH = 16
NPH = CH // PH
EPW = C * CH
EPAD = EPW * NW
PAD = EPAD - E
NACC = 10240
RPT = NACC // NS
NDISC = NACC - N


def _seg_sum_body(table, src, dst, out, src_v, dst_v, rows_v, acc, gsem, ssem):
    c = lax.axis_index("c")
    s = lax.axis_index("s")
    wid = c * NS + s

    def zrow(i, carry):
        for j in range(D // 16):
            rows_v[0, i, pl.ds(j * 16, 16)] = jnp.zeros((16,), jnp.float32)
        return carry

    lax.fori_loop(0, C, zrow, 0)
    for m in range(RPT // C):
        pltpu.sync_copy(rows_v.at[0], acc.at[pl.ds(s * RPT + m * C, C)])
    plsc.subcore_barrier()

    def wait_rows(sem):
        pltpu.make_async_copy(table.at[pl.ds(0, C)], rows_v.at[0], sem).wait()

    for ph in range(NPH):
        pltpu.sync_copy(src.at[wid, pl.ds(ph * PH, PH)], src_v)
        pltpu.sync_copy(dst.at[wid, pl.ds(ph * PH, PH)], dst_v)
        pltpu.async_copy(table.at[src_v.at[0]], rows_v.at[0], gsem)

        def body(k0, carry):
            j0 = k0 * 2
            for b in range(2):
                j = j0 + b

                @pl.when(j + 1 < PH)
                def _():
                    pltpu.async_copy(
                        table.at[src_v.at[j + 1]], rows_v.at[1 - b], gsem
                    )

                wait_rows(gsem)
                pltpu.sync_copy(rows_v.at[b], acc.at[dst_v.at[j]], add=True)
            return carry

        lax.fori_loop(0, PH // 2, body, 0)

    plsc.subcore_barrier()
    pltpu.sync_copy(acc.at[pl.ds(s * RPT, RPT)], out.at[c, pl.ds(s * RPT, RPT)])


import functools


@functools.cache
def _make_seg_sum():
    return pl.kernel(
        _seg_sum_body,
        out_type=jax.ShapeDtypeStruct((NC, NACC, D), jnp.float32),
        mesh=plsc.VectorSubcoreMesh(
            core_axis_name="c", subcore_axis_name="s", num_cores=NC, num_subcores=NS
        ),
        scratch_types=[
            pltpu.VMEM((PH, C), jnp.int32),
            pltpu.VMEM((PH, C), jnp.int32),
            pltpu.VMEM((2, C, D), jnp.float32),
            pltpu.VMEM_SHARED((NACC, D), jnp.float32),
            pltpu.SemaphoreType.DMA,
            pltpu.SemaphoreType.DMA,
        ],
    )


RB = 400
GRID = N // RB


def _dense1_body(p0, p1, x, wl1, wr1, b1, g, b, wl2, wr2, b2, y2, r2):
    agg = p0[0] + p1[0]
    h = jnp.dot(agg, wl1[...], preferred_element_type=jnp.float32)
    h = h + jnp.dot(x[...], wr1[...], preferred_element_type=jnp.float32)
    h = h + b1[...]
    mu = jnp.mean(h, axis=-1, keepdims=True)
    var = jnp.mean((h - mu) ** 2, axis=-1, keepdims=True)
    hn = (h - mu) * lax.rsqrt(var + 1e-5) * g[...] + b[...]
    hr = jnp.maximum(hn, 0.0)
    y2[...] = jnp.dot(hr, wl2[...], preferred_element_type=jnp.float32)
    r2[...] = jnp.dot(hr, wr2[...], preferred_element_type=jnp.float32) + b2[...]


def _combine_body(a, b, c, o):
    o[...] = a[0] + b[0] + c[...]


_pblk0 = pl.BlockSpec((1, RB, D), lambda i: (0, i, 0))
_pblk1 = pl.BlockSpec((1, RB, D), lambda i: (1, i, 0))


def _dense1(p, x, W_l1, W_r1, b1r, gr, br, W_l2, W_r2, b2r):
    blk = lambda r, d: pl.BlockSpec((r, d), lambda i: (i, 0))
    full = lambda a, d: pl.BlockSpec((a, d), lambda i: (0, 0))
    return pl.pallas_call(
        _dense1_body,
        grid=(GRID,),
        in_specs=[
            _pblk0, _pblk1, blk(RB, D),
            full(D, DH), full(D, DH), full(1, DH), full(1, DH), full(1, DH),
            full(DH, D), full(DH, D), full(1, D),
        ],
        out_specs=[blk(RB, D), blk(RB, D)],
        out_shape=[
            jax.ShapeDtypeStruct((N, D), jnp.float32),
            jax.ShapeDtypeStruct((N, D), jnp.float32),
        ],
    )(p, p, x, W_l1, W_r1, b1r, gr, br, W_l2, W_r2, b2r)


def _combine(q, r2):
    blk = pl.BlockSpec((RB, D), lambda i: (i, 0))
    return pl.pallas_call(
        _combine_body,
        grid=(GRID,),
        in_specs=[_pblk0, _pblk1, blk],
        out_specs=blk,
        out_shape=jax.ShapeDtypeStruct((N, D), jnp.float32),
    )(q, q, r2)


def kernel(x, edge_index, W_l1, W_r1, b1, ln_g, ln_b, W_l2, W_r2, b2):
    f32 = jnp.float32
    i32 = jnp.int32
    pad_idx = jnp.arange(PAD, dtype=i32)
    pad_src = pad_idx % N
    src = jnp.concatenate([edge_index[0], pad_src]).reshape(NW, CH, C)
    pad_dst = N + pad_idx % NDISC
    dst = jnp.concatenate([edge_index[1], pad_dst]).reshape(NW, CH, C)

    seg_sum = _make_seg_sum()
    p = seg_sum(x, src, dst)

    b1r = b1.reshape(1, DH)
    gr = ln_g.reshape(1, DH)
    br = ln_b.reshape(1, DH)
    b2r = b2.reshape(1, D)
    y2, r2 = _dense1(p, x, W_l1, W_r1, b1r, gr, br, W_l2, W_r2, b2r)

    q = seg_sum(y2, src, dst)

    return _combine(q, r2)
